# baseline (speedup 1.0000x reference)
import jax
import jax.numpy as jnp
from jax.experimental import pallas as pl

NUM_CLASSES_K = 1000
ROWS_K = 4096 * 20
BLK_K = 512


def _onehot_block(x_ref, o_ref):
    classes = jax.lax.broadcasted_iota(jnp.int32, (BLK_K, NUM_CLASSES_K), 1)
    o_ref[...] = (x_ref[...] == classes).astype(jnp.float32)


def kernel(x):
    xf = x.astype(jnp.int32).reshape(ROWS_K, 1)
    out = pl.pallas_call(
        _onehot_block,
        grid=(ROWS_K // BLK_K,),
        in_specs=[pl.BlockSpec((BLK_K, 1), lambda i: (i, 0))],
        out_specs=pl.BlockSpec((BLK_K, NUM_CLASSES_K), lambda i: (i, 0)),
        out_shape=jax.ShapeDtypeStruct((ROWS_K, NUM_CLASSES_K), jnp.float32),
    )(xf)
    return out.reshape(4096, 20, NUM_CLASSES_K)


# speedup vs baseline: 1.0641x; 1.0641x over previous
import jax
import jax.numpy as jnp
from jax.experimental import pallas as pl

NUM_CLASSES_K = 1000
ROWS_K = 4096 * 20
BLK_K = 2048


def _onehot_block(x_ref, o_ref):
    classes = jax.lax.broadcasted_iota(jnp.int32, (BLK_K, NUM_CLASSES_K), 1)
    o_ref[...] = (x_ref[...] == classes).astype(jnp.float32)


def kernel(x):
    xf = x.astype(jnp.int32).reshape(ROWS_K, 1)
    out = pl.pallas_call(
        _onehot_block,
        grid=(ROWS_K // BLK_K,),
        in_specs=[pl.BlockSpec((BLK_K, 1), lambda i: (i, 0))],
        out_specs=pl.BlockSpec((BLK_K, NUM_CLASSES_K), lambda i: (i, 0)),
        out_shape=jax.ShapeDtypeStruct((ROWS_K, NUM_CLASSES_K), jnp.float32),
    )(xf)
    return out.reshape(4096, 20, NUM_CLASSES_K)


# speedup vs baseline: 1.7389x; 1.6342x over previous
import jax
import jax.numpy as jnp
from jax.experimental import pallas as pl

NUM_CLASSES_K = 1000
ROWS_K = 4096
COLS_K = 20
BLK_K = 128


def _onehot_block(x_ref, o_ref):
    classes = jax.lax.broadcasted_iota(jnp.int32, (BLK_K, COLS_K, NUM_CLASSES_K), 2)
    o_ref[...] = (x_ref[...][:, :, None] == classes).astype(jnp.float32)


def kernel(x):
    xi = x.astype(jnp.int32)
    return pl.pallas_call(
        _onehot_block,
        grid=(ROWS_K // BLK_K,),
        in_specs=[pl.BlockSpec((BLK_K, COLS_K), lambda i: (i, 0))],
        out_specs=pl.BlockSpec((BLK_K, COLS_K, NUM_CLASSES_K), lambda i: (i, 0, 0)),
        out_shape=jax.ShapeDtypeStruct((ROWS_K, COLS_K, NUM_CLASSES_K), jnp.float32),
    )(xi)


# speedup vs baseline: 1.7423x; 1.0020x over previous
import jax
import jax.numpy as jnp
from jax.experimental import pallas as pl
from jax.experimental.pallas import tpu as pltpu

NUM_CLASSES_K = 1000
ROWS_K = 4096
COLS_K = 20
BR_K = 64
NBUF_K = 8
NSTEPS_K = ROWS_K // BR_K


def _onehot_body(x_ref, o_hbm, vbuf, sems):
    i = pl.program_id(0)
    slot = jax.lax.rem(i, NBUF_K)

    @pl.when(i >= NBUF_K)
    def _wait_prev():
        rows = (i - NBUF_K) * BR_K
        pltpu.make_async_copy(
            vbuf.at[slot],
            o_hbm.at[pl.ds(rows, BR_K)],
            sems.at[slot],
        ).wait()

    classes = jax.lax.broadcasted_iota(jnp.int32, (BR_K, COLS_K, NUM_CLASSES_K), 2)
    vbuf[slot] = (x_ref[...][:, :, None] == classes).astype(jnp.float32)

    pltpu.make_async_copy(
        vbuf.at[slot],
        o_hbm.at[pl.ds(i * BR_K, BR_K)],
        sems.at[slot],
    ).start()

    @pl.when(i == NSTEPS_K - 1)
    def _drain():
        for k in range(NBUF_K):
            j = NSTEPS_K - NBUF_K + k
            pltpu.make_async_copy(
                vbuf.at[jax.lax.rem(jnp.int32(j), NBUF_K)],
                o_hbm.at[pl.ds(j * BR_K, BR_K)],
                sems.at[jax.lax.rem(jnp.int32(j), NBUF_K)],
            ).wait()


def kernel(x):
    xi = x.astype(jnp.int32)
    return pl.pallas_call(
        _onehot_body,
        grid=(NSTEPS_K,),
        in_specs=[pl.BlockSpec((BR_K, COLS_K), lambda i: (i, 0))],
        out_specs=pl.BlockSpec(memory_space=pl.ANY),
        out_shape=jax.ShapeDtypeStruct((ROWS_K, COLS_K, NUM_CLASSES_K), jnp.float32),
        scratch_shapes=[
            pltpu.VMEM((NBUF_K, BR_K, COLS_K, NUM_CLASSES_K), jnp.float32),
            pltpu.SemaphoreType.DMA((NBUF_K,)),
        ],
    )(xi)


# speedup vs baseline: 7.6015x; 4.3629x over previous
import jax
import jax.numpy as jnp
from jax.experimental import pallas as pl

NUM_CLASSES_K = 1000
BATCH_K = 4096
COLS_K = 20


def _onehot_body(xt_ref, o_ref):
    xv = xt_ref[...]
    classes = jax.lax.broadcasted_iota(
        jnp.int32, (1, NUM_CLASSES_K, BATCH_K), 1
    )
    o_ref[...] = (xv == classes).astype(jnp.float32)


def kernel(x):
    xt = x.astype(jnp.int32).T.reshape(COLS_K, 1, BATCH_K)
    out = pl.pallas_call(
        _onehot_body,
        grid=(COLS_K,),
        in_specs=[pl.BlockSpec((1, 1, BATCH_K), lambda t: (t, 0, 0))],
        out_specs=pl.BlockSpec((1, NUM_CLASSES_K, BATCH_K), lambda t: (t, 0, 0)),
        out_shape=jax.ShapeDtypeStruct((COLS_K, NUM_CLASSES_K, BATCH_K), jnp.float32),
    )(xt)
    return out.transpose(2, 0, 1)
